# baseline (speedup 1.0000x reference)
import functools

import jax
import jax.numpy as jnp
from jax.experimental import pallas as pl
from jax.experimental.pallas import tpu as pltpu

TAU = 0.07
TOPK = 10


def _snn_kernel(hb_ref, ptb_ref, lb_ref, pi_ref, aux_ref, knn_ref,
                simT_ref, iota_ref, *, n_q, n_p, qb, pb, c_real, p_total,
                n_grp, iters_per_step):
    iq = pl.program_id(0)
    ip = pl.program_id(1)
    steps_per_grp = n_p // n_grp

    @pl.when(jnp.logical_and(iq == 0, ip == 0))
    def _init_iota():
        iota_ref[...] = jax.lax.broadcasted_iota(jnp.int32, (p_total, 128), 0)

    sim = jnp.dot(hb_ref[...], ptb_ref[...],
                  preferred_element_type=jnp.float32)
    sim_t = sim.T
    buf = iq % 2
    for g in range(n_grp):
        simT_ref[buf, g, pl.ds(ip * pb, pb), :] = \
            sim_t[:, g * 128:(g + 1) * 128]

    e = jnp.exp(sim * (1.0 / TAU))
    contrib = jnp.dot(e.astype(jnp.bfloat16), lb_ref[...],
                      preferred_element_type=jnp.float32)
    acc_old = pi_ref[...]
    pi_ref[...] = jnp.where(ip == 0, contrib, acc_old + contrib)

    @pl.when(ip == n_p - 1)
    def _finalize_pi():
        acc = pi_ref[...]
        cpad = acc.shape[1]
        iota_c = jax.lax.broadcasted_iota(jnp.int32, (qb, cpad), 1)
        den = jnp.sum(jnp.where(iota_c == c_real, acc, 0.0), axis=1,
                      keepdims=True)
        pi_ref[...] = acc / den

    prev_buf = (iq + 1) % 2
    g_dyn = ip // steps_per_grp
    phase = ip % steps_per_grp
    k_base = phase * iters_per_step
    col = g_dyn * 128
    iota_p = iota_ref[...]
    dens = aux_ref[0:1, pl.ds(col, 128)]
    for j in range(iters_per_step):
        k = k_base + j
        x = simT_ref[prev_buf, g_dyn]
        m = jnp.max(x, axis=0, keepdims=True)
        cand = jnp.where(x == m, iota_p, p_total)
        i = jnp.min(cand, axis=0, keepdims=True)
        if j == 0:
            dens = jnp.where(phase == 0, m, dens + m)
        else:
            dens = dens + m
        slab = knn_ref[:, pl.ds(col, 128)]
        row_iota = jax.lax.broadcasted_iota(jnp.int32, slab.shape, 0)
        knn_ref[:, pl.ds(col, 128)] = jnp.where(row_iota == k, i, slab)
        simT_ref[prev_buf, g_dyn] = jnp.where(iota_p == i, -jnp.inf, x)
    is_last = phase == steps_per_grp - 1
    aux_ref[0:1, pl.ds(col, 128)] = jnp.where(is_last, dens / float(TOPK),
                                              dens)


def kernel(h, peaks, labels):
    q, d = h.shape
    p_total, c = labels.shape[0], labels.shape[1]

    h_n = h / jnp.linalg.norm(h, axis=-1, keepdims=True)
    p_n = peaks / jnp.linalg.norm(peaks, axis=-1, keepdims=True)

    hb = h_n.astype(jnp.bfloat16)
    ptb = p_n.astype(jnp.bfloat16).T

    cpad = ((c + 1 + 127) // 128) * 128
    lb = jnp.pad(labels.astype(jnp.bfloat16), ((0, 0), (0, cpad - c)))
    ones_col = (jax.lax.broadcasted_iota(jnp.int32, (1, cpad), 1) == c)
    lb = jnp.where(ones_col, jnp.bfloat16(1.0), lb)

    qb = 512 if q % 512 == 0 else q
    pb = 1024 if p_total % 1024 == 0 else p_total
    n_q, n_p = q // qb, p_total // pb
    n_grp = qb // 128
    assert n_p % n_grp == 0 and (n_grp * TOPK) % n_p == 0
    iters_per_step = (n_grp * TOPK) // n_p

    body = functools.partial(_snn_kernel, n_q=n_q, n_p=n_p, qb=qb, pb=pb,
                             c_real=c, p_total=p_total, n_grp=n_grp,
                             iters_per_step=iters_per_step)
    last_q = n_q - 1
    pi_pad, aux, knn_t = pl.pallas_call(
        body,
        grid=(n_q + 1, n_p),
        in_specs=[
            pl.BlockSpec((qb, d), lambda iq, ip: (jnp.minimum(iq, last_q), 0)),
            pl.BlockSpec((d, pb), lambda iq, ip: (0, ip)),
            pl.BlockSpec((pb, cpad), lambda iq, ip: (ip, 0)),
        ],
        out_specs=[
            pl.BlockSpec((qb, cpad),
                         lambda iq, ip: (jnp.minimum(iq, last_q), 0)),
            pl.BlockSpec((8, qb), lambda iq, ip: (0, jnp.maximum(iq - 1, 0))),
            pl.BlockSpec((16, qb), lambda iq, ip: (0, jnp.maximum(iq - 1, 0))),
        ],
        out_shape=[
            jax.ShapeDtypeStruct((q, cpad), jnp.float32),
            jax.ShapeDtypeStruct((8, q), jnp.float32),
            jax.ShapeDtypeStruct((16, q), jnp.int32),
        ],
        scratch_shapes=[
            pltpu.VMEM((2, n_grp, p_total, 128), jnp.float32),
            pltpu.VMEM((p_total, 128), jnp.int32),
        ],
    )(hb, ptb, lb)

    p_i = pi_pad[:, :c]
    density = aux[0]
    knn_indices = knn_t[:TOPK].T
    return p_i, density, knn_indices
